# initial kernel scaffold (unmeasured)
import jax
import jax.numpy as jnp
from jax import lax
from jax.experimental import pallas as pl
from jax.experimental.pallas import tpu as pltpu


FB = 512


def _peer():
    return (1 - lax.axis_index("x"), lax.axis_index("y"), lax.axis_index("z"))


def _peer_barrier():
    barrier = pltpu.get_barrier_semaphore()
    pl.semaphore_signal(
        barrier, inc=1, device_id=_peer(), device_id_type=pl.DeviceIdType.MESH
    )
    pl.semaphore_wait(barrier, 1)


def _exchange_body(x_ref, a_ref, xr_ref, ar_ref, send_sems, recv_sems):
    _peer_barrier()
    rdma_x = pltpu.make_async_remote_copy(
        src_ref=x_ref,
        dst_ref=xr_ref,
        send_sem=send_sems.at[0],
        recv_sem=recv_sems.at[0],
        device_id=_peer(),
        device_id_type=pl.DeviceIdType.MESH,
    )
    rdma_a = pltpu.make_async_remote_copy(
        src_ref=a_ref,
        dst_ref=ar_ref,
        send_sem=send_sems.at[1],
        recv_sem=recv_sems.at[1],
        device_id=_peer(),
        device_id_type=pl.DeviceIdType.MESH,
    )
    rdma_x.start()
    rdma_a.start()
    rdma_x.wait()
    rdma_a.wait()


def _exchange(x, a2):
    t, d = x.shape
    return pl.pallas_call(
        _exchange_body,
        out_shape=[
            jax.ShapeDtypeStruct((t, d), x.dtype),
            jax.ShapeDtypeStruct((t, 1), a2.dtype),
        ],
        in_specs=[
            pl.BlockSpec(memory_space=pltpu.VMEM),
            pl.BlockSpec(memory_space=pltpu.VMEM),
        ],
        out_specs=[
            pl.BlockSpec(memory_space=pltpu.VMEM),
            pl.BlockSpec(memory_space=pltpu.VMEM),
        ],
        scratch_shapes=[
            pltpu.SemaphoreType.DMA((2,)),
            pltpu.SemaphoreType.DMA((2,)),
        ],
        compiler_params=pltpu.CompilerParams(collective_id=0),
    )(x, a2)


def _moe_body(x_ref, a_ref, w1_ref, w2_ref, out_ref):
    e = pl.program_id(0)
    f = pl.program_id(1)
    e_local = w1_ref.shape[0]
    del e_local
    n_local = pl.num_programs(0)
    e_global = lax.axis_index("x") * n_local + e

    @pl.when((e == 0) & (f == 0))
    def _():
        out_ref[...] = jnp.zeros_like(out_ref)

    h = jnp.maximum(
        jnp.dot(x_ref[...], w1_ref[0], preferred_element_type=jnp.float32), 0.0
    )
    contrib = jnp.dot(h, w2_ref[0], preferred_element_type=jnp.float32)
    mask = a_ref[...] == e_global
    out_ref[...] += jnp.where(mask, contrib, 0.0)


def _moe(x, a2, w1, w2):
    t, d = x.shape
    e_local, _, f_dim = w1.shape
    grid = (e_local, f_dim // FB)
    return pl.pallas_call(
        _moe_body,
        grid=grid,
        in_specs=[
            pl.BlockSpec((t, d), lambda e, f: (0, 0)),
            pl.BlockSpec((t, 1), lambda e, f: (0, 0)),
            pl.BlockSpec((1, d, FB), lambda e, f: (e, 0, f)),
            pl.BlockSpec((1, FB, d), lambda e, f: (e, f, 0)),
        ],
        out_specs=pl.BlockSpec((t, d), lambda e, f: (0, 0)),
        out_shape=jax.ShapeDtypeStruct((t, d), jnp.float32),
        compiler_params=pltpu.CompilerParams(
            dimension_semantics=("arbitrary", "arbitrary")
        ),
    )(x, a2, w1, w2)


def _combine_body(local_ref, res_ref, out_ref, send_sem, recv_sem):
    _peer_barrier()
    rdma = pltpu.make_async_remote_copy(
        src_ref=res_ref,
        dst_ref=out_ref,
        send_sem=send_sem,
        recv_sem=recv_sem,
        device_id=_peer(),
        device_id_type=pl.DeviceIdType.MESH,
    )
    rdma.start()
    rdma.wait()
    out_ref[...] += local_ref[...]


def _combine(local_acc, remote_res):
    t, d = local_acc.shape
    return pl.pallas_call(
        _combine_body,
        out_shape=jax.ShapeDtypeStruct((t, d), jnp.float32),
        in_specs=[
            pl.BlockSpec(memory_space=pltpu.VMEM),
            pl.BlockSpec(memory_space=pltpu.VMEM),
        ],
        out_specs=pl.BlockSpec(memory_space=pltpu.VMEM),
        scratch_shapes=[
            pltpu.SemaphoreType.DMA,
            pltpu.SemaphoreType.DMA,
        ],
        compiler_params=pltpu.CompilerParams(collective_id=1),
    )(local_acc, remote_res)


def kernel(x, assign, W1, W2):
    t = x.shape[0]
    a2 = assign.reshape(t, 1)
    xr, ar = _exchange(x, a2)
    local_acc = _moe(x, a2, W1, W2)
    remote_res = _moe(xr, ar, W1, W2)
    return _combine(local_acc, remote_res)


# baseline (device time: 2051184 ns/iter reference)
import jax
import jax.numpy as jnp
from jax import lax
from jax.experimental import pallas as pl
from jax.experimental.pallas import tpu as pltpu


TB = 1024
FB = 512


def _peer():
    return (1 - lax.axis_index("x"), lax.axis_index("y"), lax.axis_index("z"))


def _peer_barrier():
    barrier = pltpu.get_barrier_semaphore()
    pl.semaphore_signal(
        barrier, inc=1, device_id=_peer(), device_id_type=pl.DeviceIdType.MESH
    )
    pl.semaphore_wait(barrier, 1)


def _exchange_body(x_ref, a_ref, xr_ref, ar_ref, send_sems, recv_sems):
    _peer_barrier()
    rdma_x = pltpu.make_async_remote_copy(
        src_ref=x_ref,
        dst_ref=xr_ref,
        send_sem=send_sems.at[0],
        recv_sem=recv_sems.at[0],
        device_id=_peer(),
        device_id_type=pl.DeviceIdType.MESH,
    )
    rdma_a = pltpu.make_async_remote_copy(
        src_ref=a_ref,
        dst_ref=ar_ref,
        send_sem=send_sems.at[1],
        recv_sem=recv_sems.at[1],
        device_id=_peer(),
        device_id_type=pl.DeviceIdType.MESH,
    )
    rdma_x.start()
    rdma_a.start()
    rdma_x.wait()
    rdma_a.wait()


def _exchange(x, a2):
    t, d = x.shape
    return pl.pallas_call(
        _exchange_body,
        out_shape=[
            jax.ShapeDtypeStruct((t, d), x.dtype),
            jax.ShapeDtypeStruct((t, 1), a2.dtype),
        ],
        in_specs=[
            pl.BlockSpec(memory_space=pl.ANY),
            pl.BlockSpec(memory_space=pl.ANY),
        ],
        out_specs=[
            pl.BlockSpec(memory_space=pl.ANY),
            pl.BlockSpec(memory_space=pl.ANY),
        ],
        scratch_shapes=[
            pltpu.SemaphoreType.DMA((2,)),
            pltpu.SemaphoreType.DMA((2,)),
        ],
        compiler_params=pltpu.CompilerParams(collective_id=0),
    )(x, a2)


def _moe_body(x_ref, a_ref, w1_ref, w2_ref, out_ref):
    e = pl.program_id(1)
    f = pl.program_id(2)
    n_local = pl.num_programs(1)
    e_global = lax.axis_index("x") * n_local + e

    @pl.when((e == 0) & (f == 0))
    def _():
        out_ref[...] = jnp.zeros_like(out_ref)

    h = jnp.maximum(
        jnp.dot(x_ref[...], w1_ref[0], preferred_element_type=jnp.float32), 0.0
    )
    contrib = jnp.dot(h, w2_ref[0], preferred_element_type=jnp.float32)
    mask = a_ref[...] == e_global
    out_ref[...] += jnp.where(mask, contrib, 0.0)


def _moe(x, a2, w1, w2):
    t, d = x.shape
    e_local, _, f_dim = w1.shape
    grid = (t // TB, e_local, f_dim // FB)
    return pl.pallas_call(
        _moe_body,
        grid=grid,
        in_specs=[
            pl.BlockSpec((TB, d), lambda t, e, f: (t, 0)),
            pl.BlockSpec((TB, 1), lambda t, e, f: (t, 0)),
            pl.BlockSpec((1, d, FB), lambda t, e, f: (e, 0, f)),
            pl.BlockSpec((1, FB, d), lambda t, e, f: (e, f, 0)),
        ],
        out_specs=pl.BlockSpec((TB, d), lambda t, e, f: (t, 0)),
        out_shape=jax.ShapeDtypeStruct((t, d), jnp.float32),
        compiler_params=pltpu.CompilerParams(
            dimension_semantics=("arbitrary", "arbitrary", "arbitrary"),
            vmem_limit_bytes=63 * 1024 * 1024,
        ),
    )(x, a2, w1, w2)


def _return_body(res_ref, out_ref, send_sem, recv_sem):
    _peer_barrier()
    rdma = pltpu.make_async_remote_copy(
        src_ref=res_ref,
        dst_ref=out_ref,
        send_sem=send_sem,
        recv_sem=recv_sem,
        device_id=_peer(),
        device_id_type=pl.DeviceIdType.MESH,
    )
    rdma.start()
    rdma.wait()


def _return(remote_res):
    t, d = remote_res.shape
    return pl.pallas_call(
        _return_body,
        out_shape=jax.ShapeDtypeStruct((t, d), jnp.float32),
        in_specs=[pl.BlockSpec(memory_space=pl.ANY)],
        out_specs=pl.BlockSpec(memory_space=pl.ANY),
        scratch_shapes=[
            pltpu.SemaphoreType.DMA,
            pltpu.SemaphoreType.DMA,
        ],
        compiler_params=pltpu.CompilerParams(collective_id=1),
    )(remote_res)


def kernel(x, assign, W1, W2):
    t = x.shape[0]
    a2 = assign.reshape(t, 1)
    xr, ar = _exchange(x, a2)
    local_acc = _moe(x, a2, W1, W2)
    remote_res = _moe(xr, ar, W1, W2)
    recv = _return(remote_res)
    return local_acc + recv


# device time: 1940809 ns/iter; 1.0569x vs baseline; 1.0569x over previous
import jax
import jax.numpy as jnp
from jax import lax
from jax.experimental import pallas as pl
from jax.experimental.pallas import tpu as pltpu


C = 640
FB = 256
N_EXPERTS = 8


def _peer():
    return (1 - lax.axis_index("x"), lax.axis_index("y"), lax.axis_index("z"))


def _peer_barrier():
    barrier = pltpu.get_barrier_semaphore()
    pl.semaphore_signal(
        barrier, inc=1, device_id=_peer(), device_id_type=pl.DeviceIdType.MESH
    )
    pl.semaphore_wait(barrier, 1)


def _xchg_body(src_ref, dst_ref, send_sem, recv_sem):
    _peer_barrier()
    rdma = pltpu.make_async_remote_copy(
        src_ref=src_ref,
        dst_ref=dst_ref,
        send_sem=send_sem,
        recv_sem=recv_sem,
        device_id=_peer(),
        device_id_type=pl.DeviceIdType.MESH,
    )
    rdma.start()
    rdma.wait()


def _xchg(buf, collective_id):
    return pl.pallas_call(
        _xchg_body,
        out_shape=jax.ShapeDtypeStruct(buf.shape, buf.dtype),
        in_specs=[pl.BlockSpec(memory_space=pl.ANY)],
        out_specs=pl.BlockSpec(memory_space=pl.ANY),
        scratch_shapes=[pltpu.SemaphoreType.DMA, pltpu.SemaphoreType.DMA],
        compiler_params=pltpu.CompilerParams(collective_id=collective_id),
    )(buf)


def _moe_body(x_ref, w1_ref, w2_ref, out_ref):
    f = pl.program_id(1)

    @pl.when(f == 0)
    def _():
        out_ref[...] = jnp.zeros_like(out_ref)

    h = jnp.maximum(
        jnp.dot(x_ref[0], w1_ref[0], preferred_element_type=jnp.float32), 0.0
    )
    out_ref[0] += jnp.dot(h, w2_ref[0], preferred_element_type=jnp.float32)


def _moe(xin, w1, w2):
    e_local, rows, d = xin.shape
    _, _, f_dim = w1.shape
    grid = (e_local, f_dim // FB)
    return pl.pallas_call(
        _moe_body,
        grid=grid,
        in_specs=[
            pl.BlockSpec((1, rows, d), lambda e, f: (e, 0, 0)),
            pl.BlockSpec((1, d, FB), lambda e, f: (e, 0, f)),
            pl.BlockSpec((1, FB, d), lambda e, f: (e, f, 0)),
        ],
        out_specs=pl.BlockSpec((1, rows, d), lambda e, f: (e, 0, 0)),
        out_shape=jax.ShapeDtypeStruct((e_local, rows, d), jnp.float32),
        compiler_params=pltpu.CompilerParams(
            dimension_semantics=("arbitrary", "arbitrary"),
            vmem_limit_bytes=63 * 1024 * 1024,
        ),
    )(xin, w1, w2)


def kernel(x, assign, W1, W2):
    t, d = x.shape
    e_local = W1.shape[0]
    sx = lax.axis_index("x")

    order = jnp.argsort(assign).astype(jnp.int32)
    sorted_assign = jnp.take(assign, order)
    counts = jnp.zeros((N_EXPERTS,), jnp.int32).at[assign].add(1)
    starts = jnp.cumsum(counts) - counts
    rank = jnp.arange(t, dtype=jnp.int32) - jnp.take(starts, sorted_assign)
    idx = jnp.full((N_EXPERTS, C + 1), t, dtype=jnp.int32)
    idx = idx.at[sorted_assign, jnp.minimum(rank, C)].set(order)
    idx = idx[:, :C]

    idx_mine = lax.dynamic_slice_in_dim(idx, sx * e_local, e_local, axis=0)
    idx_peer = lax.dynamic_slice_in_dim(
        idx, (1 - sx) * e_local, e_local, axis=0
    )
    xpad = jnp.concatenate([x, jnp.zeros((1, d), x.dtype)], axis=0)
    own_buckets = jnp.take(xpad, idx_mine, axis=0)
    send_buckets = jnp.take(xpad, idx_peer, axis=0)

    recv_buckets = _xchg(send_buckets, collective_id=0)

    xin = jnp.concatenate([own_buckets, recv_buckets], axis=1)
    res = _moe(xin, W1, W2)
    res_own = res[:, :C, :]
    res_back = res[:, C:, :]

    res_recv = _xchg(res_back, collective_id=1)

    out = jnp.zeros((t + 1, d), jnp.float32)
    out = out.at[idx_mine.reshape(-1)].set(res_own.reshape(-1, d))
    out = out.at[idx_peer.reshape(-1)].set(res_recv.reshape(-1, d))
    return out[:t]


# device time: 847788 ns/iter; 2.4195x vs baseline; 2.2893x over previous
import jax
import jax.numpy as jnp
from jax import lax
from jax.experimental import pallas as pl
from jax.experimental.pallas import tpu as pltpu


TB = 1024
FB = 512


def _x_peer():
    return (1 - lax.axis_index("x"), lax.axis_index("y"), lax.axis_index("z"))


def _barrier(partners):
    barrier = pltpu.get_barrier_semaphore()
    for p in partners:
        pl.semaphore_signal(
            barrier, inc=1, device_id=p, device_id_type=pl.DeviceIdType.MESH
        )
    pl.semaphore_wait(barrier, len(partners))


def _exchange_body(x_ref, a_ref, xr_ref, ar_ref, send_sems, recv_sems):
    _barrier([_x_peer()])
    rdma_x = pltpu.make_async_remote_copy(
        src_ref=x_ref,
        dst_ref=xr_ref,
        send_sem=send_sems.at[0],
        recv_sem=recv_sems.at[0],
        device_id=_x_peer(),
        device_id_type=pl.DeviceIdType.MESH,
    )
    rdma_a = pltpu.make_async_remote_copy(
        src_ref=a_ref,
        dst_ref=ar_ref,
        send_sem=send_sems.at[1],
        recv_sem=recv_sems.at[1],
        device_id=_x_peer(),
        device_id_type=pl.DeviceIdType.MESH,
    )
    rdma_x.start()
    rdma_a.start()
    rdma_x.wait()
    rdma_a.wait()


def _exchange(x, a2):
    t, d = x.shape
    return pl.pallas_call(
        _exchange_body,
        out_shape=[
            jax.ShapeDtypeStruct((t, d), x.dtype),
            jax.ShapeDtypeStruct((t, 1), a2.dtype),
        ],
        in_specs=[
            pl.BlockSpec(memory_space=pl.ANY),
            pl.BlockSpec(memory_space=pl.ANY),
        ],
        out_specs=[
            pl.BlockSpec(memory_space=pl.ANY),
            pl.BlockSpec(memory_space=pl.ANY),
        ],
        scratch_shapes=[
            pltpu.SemaphoreType.DMA((2,)),
            pltpu.SemaphoreType.DMA((2,)),
        ],
        compiler_params=pltpu.CompilerParams(collective_id=0),
    )(x, a2)


def _moe_body(x_ref, a_ref, w1_ref, w2_ref, out_ref):
    e = pl.program_id(0)
    f = pl.program_id(1)
    n_local = pl.num_programs(0)
    e_global = lax.axis_index("x") * n_local + e

    @pl.when((e == 0) & (f == 0))
    def _():
        out_ref[...] = jnp.zeros_like(out_ref)

    h = jnp.maximum(
        jnp.dot(x_ref[...], w1_ref[0], preferred_element_type=jnp.float32), 0.0
    )
    contrib = jnp.dot(h, w2_ref[0], preferred_element_type=jnp.float32)
    mask = a_ref[...] == e_global
    out_ref[...] += jnp.where(mask, contrib, 0.0)


def _moe(x, a2, w1, w2):
    t, d = x.shape
    e_local, _, f_dim = w1.shape
    grid = (e_local, f_dim // FB)
    return pl.pallas_call(
        _moe_body,
        grid=grid,
        in_specs=[
            pl.BlockSpec((t, d), lambda e, f: (0, 0)),
            pl.BlockSpec((t, 1), lambda e, f: (0, 0)),
            pl.BlockSpec((1, d, FB), lambda e, f: (e, 0, f)),
            pl.BlockSpec((1, FB, d), lambda e, f: (e, f, 0)),
        ],
        out_specs=pl.BlockSpec((t, d), lambda e, f: (0, 0)),
        out_shape=jax.ShapeDtypeStruct((t, d), jnp.float32),
        compiler_params=pltpu.CompilerParams(
            dimension_semantics=("arbitrary", "arbitrary"),
            vmem_limit_bytes=63 * 1024 * 1024,
        ),
    )(x, a2, w1, w2)


def _return_body(res_ref, out_ref, send_sem, recv_sem):
    _barrier([_x_peer()])
    rdma = pltpu.make_async_remote_copy(
        src_ref=res_ref,
        dst_ref=out_ref,
        send_sem=send_sem,
        recv_sem=recv_sem,
        device_id=_x_peer(),
        device_id_type=pl.DeviceIdType.MESH,
    )
    rdma.start()
    rdma.wait()


def _return(remote_res):
    t, d = remote_res.shape
    return pl.pallas_call(
        _return_body,
        out_shape=jax.ShapeDtypeStruct((t, d), jnp.float32),
        in_specs=[pl.BlockSpec(memory_space=pl.ANY)],
        out_specs=pl.BlockSpec(memory_space=pl.ANY),
        scratch_shapes=[pltpu.SemaphoreType.DMA, pltpu.SemaphoreType.DMA],
        compiler_params=pltpu.CompilerParams(collective_id=1),
    )(remote_res)


def _ag_body(chunk_ref, out_ref, local_sem, send_sems, recv_sems):
    sx = lax.axis_index("x")
    sy = lax.axis_index("y")
    sz = lax.axis_index("z")
    q = 2 * sy + sz
    z_peer = (sx, sy, 1 - sz)
    y_peer = (sx, 1 - sy, sz)
    _barrier([z_peer, y_peer])

    cp = pltpu.make_async_copy(
        chunk_ref, out_ref.at[pl.ds(q * TB, TB), :], local_sem
    )
    cp.start()
    cp.wait()

    r1 = pltpu.make_async_remote_copy(
        src_ref=out_ref.at[pl.ds(q * TB, TB), :],
        dst_ref=out_ref.at[pl.ds(q * TB, TB), :],
        send_sem=send_sems.at[0],
        recv_sem=recv_sems.at[0],
        device_id=z_peer,
        device_id_type=pl.DeviceIdType.MESH,
    )
    r1.start()
    r1.wait()

    r2 = pltpu.make_async_remote_copy(
        src_ref=out_ref.at[pl.ds(sy * 2 * TB, 2 * TB), :],
        dst_ref=out_ref.at[pl.ds(sy * 2 * TB, 2 * TB), :],
        send_sem=send_sems.at[1],
        recv_sem=recv_sems.at[1],
        device_id=y_peer,
        device_id_type=pl.DeviceIdType.MESH,
    )
    r2.start()
    r2.wait()


def _ag(chunk_out, t_full):
    t, d = chunk_out.shape
    return pl.pallas_call(
        _ag_body,
        out_shape=jax.ShapeDtypeStruct((t_full, d), jnp.float32),
        in_specs=[pl.BlockSpec(memory_space=pl.ANY)],
        out_specs=pl.BlockSpec(memory_space=pl.ANY),
        scratch_shapes=[
            pltpu.SemaphoreType.DMA,
            pltpu.SemaphoreType.DMA((2,)),
            pltpu.SemaphoreType.DMA((2,)),
        ],
        compiler_params=pltpu.CompilerParams(collective_id=2),
    )(chunk_out)


def kernel(x, assign, W1, W2):
    t, d = x.shape
    sy = lax.axis_index("y")
    sz = lax.axis_index("z")
    q = 2 * sy + sz

    xc = lax.dynamic_slice_in_dim(x, q * TB, TB, axis=0)
    ac = lax.dynamic_slice_in_dim(assign, q * TB, TB, axis=0).reshape(TB, 1)

    xr, ar = _exchange(xc, ac)
    local_acc = _moe(xc, ac, W1, W2)
    remote_res = _moe(xr, ar, W1, W2)
    recv = _return(remote_res)
    chunk_out = local_acc + recv
    return _ag(chunk_out, t)
